# baseline (speedup 1.0000x reference)
import jax
import jax.numpy as jnp
from jax import lax
from jax.experimental import pallas as pl
from jax.experimental.pallas import tpu as pltpu
from jax.experimental.pallas import tpu_sc as plsc

N = 10000
E = 320000
R = 16
HID = 128
HALF = 64
NSUB = 16
EPW = E // NSUB
K = 80
NCH = EPW // K
EPS_BN = 1e-5
EPS_AGG = 1e-6
G = HALF // 16


def _sigmoid16(x):
    return 1.0 / (1.0 + jnp.exp(-x))


def _sc_edge_pass(pass1):
    def body(*refs):
        if pass1:
            (src_h, dst_h, et_h, hA, hB, hV, eC,
             acc_o, eh_o, st_o,
             is0, id0, ie0, is1, id1, ie1,
             as0, ad0, ae0, as1, ad1, ae1, sdst,
             av0, av1, bv0, bv1, ev, vv,
             comb_v, eh_v, st_v,
             acc_sh, sg0, sg1, se, si0, si1, ss, sh) = refs
            z_h = None
            zv = None
        else:
            (src_h, dst_h, et_h, hA, hB, hV, eC, z_h,
             acc_o,
             is0, id0, ie0, is1, id1, ie1,
             as0, ad0, ae0, as1, ad1, ae1, sdst,
             av0, av1, bv0, bv1, ev, vv, zv,
             comb_v,
             acc_sh, sg0, sg1, se, si0, si1, ss, sh) = refs

        isrc, idst, iet = (is0, is1), (id0, id1), (ie0, ie1)
        asrc, adst, aet = (as0, as1), (ad0, ad1), (ae0, ae1)
        av, bv = (av0, av1), (bv0, bv1)
        semg, semi = (sg0, sg1), (si0, si1)

        c = lax.axis_index("c")
        s = lax.axis_index("s")
        base = s * EPW
        coff = c * N
        eoff = c * R

        zero = jnp.zeros((16,), jnp.float32)

        def zb_body(i, _):
            for g in range(2 * G):
                comb_v[i, pl.ds(g * 16, 16)] = zero
            return 0
        lax.fori_loop(0, 40, zb_body, 0)

        @pl.when(s < 10)
        def _zero_shared():
            zrows = comb_v.at[pl.ds(0, 40)]
            for r5 in range(25):
                pltpu.sync_copy(zrows, acc_sh.at[pl.ds(s * 1000 + r5 * 40, 40)])
        plsc.subcore_barrier()

        def fire_idx(j, p):
            off = base + j * K
            pltpu.async_copy(src_h.at[pl.ds(off, K)], isrc[p], semi[p])
            pltpu.async_copy(dst_h.at[pl.ds(off, K)], idst[p], semi[p])
            pltpu.async_copy(et_h.at[pl.ds(off, K)], iet[p], semi[p])

        def wait_idx(p):
            pltpu.make_async_copy(src_h.at[pl.ds(0, K)], isrc[p], semi[p]).wait()
            pltpu.make_async_copy(dst_h.at[pl.ds(0, K)], idst[p], semi[p]).wait()
            pltpu.make_async_copy(et_h.at[pl.ds(0, K)], iet[p], semi[p]).wait()

        def adjust(p):
            for q in range(K // 16):
                sl = pl.ds(q * 16, 16)
                asrc[p][sl] = isrc[p][sl] + coff
                adst[p][sl] = idst[p][sl] + coff
                aet[p][sl] = iet[p][sl] + eoff

        def fire_ab(p):
            pltpu.async_copy(hA.at[asrc[p]], av[p], semg[p])
            pltpu.async_copy(hB.at[adst[p]], bv[p], semg[p])

        def wait_ab(p):
            pltpu.make_async_copy(hA.at[asrc[p]], av[p], semg[p]).wait()
            pltpu.make_async_copy(hB.at[adst[p]], bv[p], semg[p]).wait()

        def fire_ev(j, p):
            pltpu.async_copy(eC.at[aet[p]], ev, se)
            pltpu.async_copy(hV.at[asrc[p]], vv, se)
            if not pass1:
                off = base + j * K
                pltpu.async_copy(z_h.at[c, pl.ds(off, K)], zv, se)

        def wait_ev(p):
            pltpu.make_async_copy(eC.at[aet[p]], ev, se).wait()
            pltpu.make_async_copy(hV.at[asrc[p]], vv, se).wait()
            if not pass1:
                pltpu.make_async_copy(z_h.at[c, pl.ds(0, K)], zv, se).wait()

        def wait_scatter():
            pltpu.make_async_copy(comb_v, acc_sh.at[sdst], ss).wait()

        def wait_eh(j):
            pltpu.make_async_copy(eh_v, eh_o.at[c, pl.ds(0, K)], sh).wait()

        def compute(p, stats):
            a_v, b_v, e_v, v_v, z_v = av[p], bv[p], ev, vv, zv

            def edge2(i2, st):
                out = list(st) if pass1 else st
                for d in range(2):
                    i = i2 * 2 + d
                    for g in range(G):
                        sl = pl.ds(g * 16, 16)
                        sl2 = pl.ds(HALF + g * 16, 16)
                        eh = a_v[i, sl] + b_v[i, sl] + e_v[i, sl]
                        if not pass1:
                            eh = eh + z_v[i, sl]
                        sg = _sigmoid16(eh)
                        comb_v[i, sl2] = sg
                        comb_v[i, sl] = sg * v_v[i, sl]
                        if pass1:
                            eh_v[i, sl] = eh
                            out[g] = out[g] + eh
                            out[G + g] = out[G + g] + eh * eh
                return tuple(out) if pass1 else out
            return lax.fori_loop(0, K // 2, edge2, stats)

        def step(j, p, stats):
            q = 1 - p

            @pl.when(j + 1 < NCH)
            def _prefetch():
                wait_idx(q)
                adjust(q)
                fire_ab(q)

            wait_ab(p)
            wait_ev(p)
            if pass1:
                @pl.when(j > 0)
                def _weh():
                    wait_eh(j - 1)

            @pl.when(j > 0)
            def _wsc():
                wait_scatter()

            stats = compute(p, stats)

            @pl.when(j + 1 < NCH)
            def _fire_ev_next():
                fire_ev(j + 1, q)

            for q16 in range(K // 16):
                sl = pl.ds(q16 * 16, 16)
                sdst[sl] = idst[p][sl]
            pltpu.async_copy(comb_v, acc_sh.at[sdst], ss, add=True)
            if pass1:
                pltpu.async_copy(eh_v, eh_o.at[c, pl.ds(base + j * K, K)], sh)

            @pl.when(j + 2 < NCH)
            def _next_idx():
                fire_idx(j + 2, p)
            return stats

        fire_idx(0, 0)
        wait_idx(0)
        adjust(0)
        fire_ab(0)
        fire_ev(0, 0)
        fire_idx(1, 1)

        stats0 = tuple(jnp.zeros((16,), jnp.float32) for _ in range(2 * G)) if pass1 else 0

        def pair(i, stats):
            j = i * 2
            stats = step(j, 0, stats)
            stats = step(j + 1, 1, stats)
            return stats
        stats = lax.fori_loop(0, NCH // 2, pair, stats0)

        wait_scatter()
        if pass1:
            wait_eh(NCH - 1)
            for g in range(G):
                st_v[0, pl.ds(g * 16, 16)] = stats[g]
                st_v[1, pl.ds(g * 16, 16)] = stats[G + g]
            pltpu.sync_copy(st_v, st_o.at[c, s])

        plsc.subcore_barrier()

        @pl.when(s < 10)
        def _copy_out():
            rows = pl.ds(s * 1000, 1000)
            pltpu.sync_copy(acc_sh.at[rows], acc_o.at[c, rows])

    return body


def _idx_bufs():
    return [pltpu.VMEM((K,), jnp.int32) for _ in range(13)]


def _row_bufs(n):
    return [pltpu.VMEM((K, HALF), jnp.float32) for _ in range(n)]


_SEMS = [pltpu.SemaphoreType.DMA] * 7


def _make_sc1():
    mesh = plsc.VectorSubcoreMesh(core_axis_name="c", subcore_axis_name="s")
    return pl.kernel(
        _sc_edge_pass(True),
        out_type=[
            jax.ShapeDtypeStruct((2, N, HID), jnp.float32),
            jax.ShapeDtypeStruct((2, E, HALF), jnp.float32),
            jax.ShapeDtypeStruct((2, NSUB, 2, HALF), jnp.float32),
        ],
        mesh=mesh,
        scratch_types=(
            _idx_bufs() + _row_bufs(6)
            + [pltpu.VMEM((K, HID), jnp.float32),
               pltpu.VMEM((K, HALF), jnp.float32),
               pltpu.VMEM((2, HALF), jnp.float32),
               pltpu.VMEM_SHARED((N, HID), jnp.float32)]
            + _SEMS
        ),
        compiler_params=pltpu.CompilerParams(use_tc_tiling_on_sc=False),
    )


def _make_sc2():
    mesh = plsc.VectorSubcoreMesh(core_axis_name="c", subcore_axis_name="s")
    return pl.kernel(
        _sc_edge_pass(False),
        out_type=[
            jax.ShapeDtypeStruct((2, N, HID), jnp.float32),
        ],
        mesh=mesh,
        scratch_types=(
            _idx_bufs() + _row_bufs(7)
            + [pltpu.VMEM((K, HID), jnp.float32),
               pltpu.VMEM_SHARED((N, HID), jnp.float32)]
            + _SEMS
        ),
        compiler_params=pltpu.CompilerParams(use_tc_tiling_on_sc=False),
    )


def _tc_tables(h_ref, e_ref, A_ref, B_ref, V_ref, U_ref, C_ref,
               hA_ref, hB_ref, hV_ref, hU_ref, eC_ref):
    h = h_ref[...]
    for w_ref, o_ref in ((A_ref, hA_ref), (B_ref, hB_ref), (V_ref, hV_ref)):
        x = jnp.dot(h, w_ref[...], preferred_element_type=jnp.float32)
        o_ref[0] = x[:, :HALF]
        o_ref[1] = x[:, HALF:]
    hU_ref[...] = jnp.dot(h, U_ref[...], preferred_element_type=jnp.float32)
    ec = jnp.dot(e_ref[...], C_ref[...], preferred_element_type=jnp.float32)
    eC_ref[0] = ec[:, :HALF]
    eC_ref[1] = ec[:, HALF:]


_tables_call = pl.pallas_call(
    _tc_tables,
    out_shape=[
        jax.ShapeDtypeStruct((2, N, HALF), jnp.float32),
        jax.ShapeDtypeStruct((2, N, HALF), jnp.float32),
        jax.ShapeDtypeStruct((2, N, HALF), jnp.float32),
        jax.ShapeDtypeStruct((N, HID), jnp.float32),
        jax.ShapeDtypeStruct((2, R, HALF), jnp.float32),
    ],
)


def _tc_hupdate(h_ref, hU_ref, acc_ref, h1_ref):
    num = jnp.concatenate([acc_ref[0, :, :HALF], acc_ref[1, :, :HALF]], axis=1)
    den = jnp.concatenate([acc_ref[0, :, HALF:], acc_ref[1, :, HALF:]], axis=1)
    x = hU_ref[...] + num / (den + EPS_AGG)
    mu = jnp.mean(x, axis=0, keepdims=True)
    var = jnp.mean((x - mu) * (x - mu), axis=0, keepdims=True)
    hn = jnp.maximum((x - mu) / jnp.sqrt(var + EPS_BN), 0.0)
    h1_ref[...] = h_ref[...] + hn


_hupdate_call = pl.pallas_call(
    _tc_hupdate,
    out_shape=jax.ShapeDtypeStruct((N, HID), jnp.float32),
)

BM = 512


def _tc_edge_mm(mu_ref, inv_ref, C_ref, eh_ref, z_ref):
    lo = eh_ref[0]
    hi = eh_ref[1]
    t_lo = jnp.maximum((lo - mu_ref[0:1, :HALF]) * inv_ref[0:1, :HALF], 0.0)
    t_hi = jnp.maximum((hi - mu_ref[0:1, HALF:]) * inv_ref[0:1, HALF:], 0.0)
    z = (jnp.dot(t_lo, C_ref[:HALF, :], preferred_element_type=jnp.float32)
         + jnp.dot(t_hi, C_ref[HALF:, :], preferred_element_type=jnp.float32))
    z_ref[0] = z[:, :HALF]
    z_ref[1] = z[:, HALF:]


_edge_mm_call = pl.pallas_call(
    _tc_edge_mm,
    grid=(E // BM,),
    in_specs=[
        pl.BlockSpec((1, HID), lambda i: (0, 0)),
        pl.BlockSpec((1, HID), lambda i: (0, 0)),
        pl.BlockSpec((HID, HID), lambda i: (0, 0)),
        pl.BlockSpec((2, BM, HALF), lambda i: (0, i, 0)),
    ],
    out_specs=pl.BlockSpec((2, BM, HALF), lambda i: (0, i, 0)),
    out_shape=jax.ShapeDtypeStruct((2, E, HALF), jnp.float32),
)


def kernel(edge_index, node_id, edge_type, h_emb, e_emb, A, B, C, U, V):
    src = edge_index[0].astype(jnp.int32)
    dst = edge_index[1].astype(jnp.int32)
    et = edge_type.astype(jnp.int32)
    h = h_emb

    sc1 = _make_sc1()
    sc2 = _make_sc2()

    hA0, hB0, hV0, hU0, eC0 = _tables_call(h, e_emb, A[0], B[0], V[0], U[0], C[0])
    acc0, eh0, st0 = sc1(
        src, dst, et,
        hA0.reshape(2 * N, HALF), hB0.reshape(2 * N, HALF),
        hV0.reshape(2 * N, HALF), eC0.reshape(2 * R, HALF))

    ssum = jnp.sum(st0, axis=1)
    mu = jnp.concatenate([ssum[0, 0], ssum[1, 0]]) * (1.0 / E)
    ex2 = jnp.concatenate([ssum[0, 1], ssum[1, 1]]) * (1.0 / E)
    inv = 1.0 / jnp.sqrt(jnp.maximum(ex2 - mu * mu, 0.0) + EPS_BN)
    mu = mu.reshape(1, HID)
    inv = inv.reshape(1, HID)

    h1 = _hupdate_call(h, hU0, acc0)
    hA1, hB1, hV1, hU1, eC1 = _tables_call(h1, e_emb, A[1], B[1], V[1], U[1], C[1])
    z = _edge_mm_call(mu, inv, C[1], eh0)

    (acc1,) = sc2(
        src, dst, et,
        hA1.reshape(2 * N, HALF), hB1.reshape(2 * N, HALF),
        hV1.reshape(2 * N, HALF), eC1.reshape(2 * R, HALF),
        z)
    return _hupdate_call(h1, hU1, acc1)


# speedup vs baseline: 1.0065x; 1.0065x over previous
import jax
import jax.numpy as jnp
from jax import lax
from jax.experimental import pallas as pl
from jax.experimental.pallas import tpu as pltpu
from jax.experimental.pallas import tpu_sc as plsc

N = 10000
E = 320000
R = 16
HID = 128
HALF = 64
NSUB = 16
EPW = E // NSUB
K = 80
NCH = EPW // K
EPS_BN = 1e-5
EPS_AGG = 1e-6
G = HALF // 16


def _sigmoid16(x):
    return 1.0 / (1.0 + jnp.exp(-x))


def _sc_edge_pass(pass1):
    def body(*refs):
        if pass1:
            (src_h, dst_h, et_h, hA, hB, hV, eC,
             acc_o, eh_o, st_o,
             is0, id0, ie0, is1, id1, ie1,
             as0, ad0, ae0, as1, ad1, ae1, sdst,
             av0, av1, bv0, bv1, ev, vv,
             comb_v, eh_v, st_v,
             acc_sh, sg0, sg1, se, si0, si1, ss, sh) = refs
            z_h = None
            zv = None
        else:
            (src_h, dst_h, et_h, hA, hB, hV, eC, z_h,
             acc_o,
             is0, id0, ie0, is1, id1, ie1,
             as0, ad0, ae0, as1, ad1, ae1, sdst,
             av0, av1, bv0, bv1, ev, vv, zv,
             comb_v,
             acc_sh, sg0, sg1, se, si0, si1, ss, sh) = refs

        isrc, idst, iet = (is0, is1), (id0, id1), (ie0, ie1)
        asrc, adst, aet = (as0, as1), (ad0, ad1), (ae0, ae1)
        av, bv = (av0, av1), (bv0, bv1)
        semg, semi = (sg0, sg1), (si0, si1)

        c = lax.axis_index("c")
        s = lax.axis_index("s")
        base = s * EPW
        coff = c * N
        eoff = c * R

        zero = jnp.zeros((16,), jnp.float32)

        def zb_body(i, _):
            for g in range(2 * G):
                comb_v[i, pl.ds(g * 16, 16)] = zero
            return 0
        lax.fori_loop(0, 40, zb_body, 0)

        @pl.when(s < 10)
        def _zero_shared():
            zrows = comb_v.at[pl.ds(0, 40)]
            for r5 in range(25):
                pltpu.sync_copy(zrows, acc_sh.at[pl.ds(s * 1000 + r5 * 40, 40)])
        plsc.subcore_barrier()

        def fire_idx(j, p):
            off = base + j * K
            pltpu.async_copy(src_h.at[pl.ds(off, K)], isrc[p], semi[p])
            pltpu.async_copy(dst_h.at[pl.ds(off, K)], idst[p], semi[p])
            pltpu.async_copy(et_h.at[pl.ds(off, K)], iet[p], semi[p])

        def wait_idx(p):
            pltpu.make_async_copy(src_h.at[pl.ds(0, K)], isrc[p], semi[p]).wait()
            pltpu.make_async_copy(dst_h.at[pl.ds(0, K)], idst[p], semi[p]).wait()
            pltpu.make_async_copy(et_h.at[pl.ds(0, K)], iet[p], semi[p]).wait()

        def adjust(p):
            for q in range(K // 16):
                sl = pl.ds(q * 16, 16)
                asrc[p][sl] = isrc[p][sl] + coff
                adst[p][sl] = idst[p][sl] + coff
                aet[p][sl] = iet[p][sl] + eoff

        def fire_ab(p):
            pltpu.async_copy(hA.at[asrc[p]], av[p], semg[p])
            pltpu.async_copy(hB.at[adst[p]], bv[p], semg[p])

        def wait_ab(p):
            pltpu.make_async_copy(hA.at[asrc[p]], av[p], semg[p]).wait()
            pltpu.make_async_copy(hB.at[adst[p]], bv[p], semg[p]).wait()

        def fire_ev(j, p):
            pltpu.async_copy(eC.at[aet[p]], ev, se)
            pltpu.async_copy(hV.at[asrc[p]], vv, se)
            if not pass1:
                off = base + j * K
                pltpu.async_copy(z_h.at[c, pl.ds(off, K)], zv, se)

        def wait_ev(p):
            pltpu.make_async_copy(eC.at[aet[p]], ev, se).wait()
            pltpu.make_async_copy(hV.at[asrc[p]], vv, se).wait()
            if not pass1:
                pltpu.make_async_copy(z_h.at[c, pl.ds(0, K)], zv, se).wait()

        def wait_scatter():
            pltpu.make_async_copy(comb_v, acc_sh.at[sdst], ss).wait()

        def wait_eh(j):
            pltpu.make_async_copy(eh_v, eh_o.at[c, pl.ds(0, K)], sh).wait()

        def compute(p, stats):
            a_v, b_v, e_v, v_v, z_v = av[p], bv[p], ev, vv, zv

            def edge2(i2, st):
                out = list(st) if pass1 else st
                for d in range(2):
                    i = i2 * 2 + d
                    for g in range(G):
                        sl = pl.ds(g * 16, 16)
                        sl2 = pl.ds(HALF + g * 16, 16)
                        eh = a_v[i, sl] + b_v[i, sl] + e_v[i, sl]
                        if not pass1:
                            eh = eh + z_v[i, sl]
                        sg = _sigmoid16(eh)
                        comb_v[i, sl2] = sg
                        comb_v[i, sl] = sg * v_v[i, sl]
                        if pass1:
                            eh_v[i, sl] = eh
                            out[g] = out[g] + eh
                            out[G + g] = out[G + g] + eh * eh
                return tuple(out) if pass1 else out
            return lax.fori_loop(0, K // 2, edge2, stats)

        def step(j, p, stats):
            q = 1 - p

            @pl.when(j + 1 < NCH)
            def _prefetch():
                wait_idx(q)
                adjust(q)
                fire_ab(q)

            wait_ab(p)
            wait_ev(p)
            if pass1:
                @pl.when(j > 0)
                def _weh():
                    wait_eh(j - 1)

            stats = compute(p, stats)

            @pl.when(j + 1 < NCH)
            def _fire_ev_next():
                fire_ev(j + 1, q)

            pltpu.sync_copy(comb_v, acc_sh.at[idst[p]], add=True)
            if pass1:
                pltpu.async_copy(eh_v, eh_o.at[c, pl.ds(base + j * K, K)], sh)

            @pl.when(j + 2 < NCH)
            def _next_idx():
                fire_idx(j + 2, p)
            return stats

        fire_idx(0, 0)
        wait_idx(0)
        adjust(0)
        fire_ab(0)
        fire_ev(0, 0)
        fire_idx(1, 1)

        stats0 = tuple(jnp.zeros((16,), jnp.float32) for _ in range(2 * G)) if pass1 else 0

        def pair(i, stats):
            j = i * 2
            stats = step(j, 0, stats)
            stats = step(j + 1, 1, stats)
            return stats
        stats = lax.fori_loop(0, NCH // 2, pair, stats0)

        if pass1:
            wait_eh(NCH - 1)
            for g in range(G):
                st_v[0, pl.ds(g * 16, 16)] = stats[g]
                st_v[1, pl.ds(g * 16, 16)] = stats[G + g]
            pltpu.sync_copy(st_v, st_o.at[c, s])

        plsc.subcore_barrier()

        @pl.when(s < 10)
        def _copy_out():
            rows = pl.ds(s * 1000, 1000)
            pltpu.sync_copy(acc_sh.at[rows], acc_o.at[c, rows])

    return body


def _idx_bufs():
    return [pltpu.VMEM((K,), jnp.int32) for _ in range(13)]


def _row_bufs(n):
    return [pltpu.VMEM((K, HALF), jnp.float32) for _ in range(n)]


_SEMS = [pltpu.SemaphoreType.DMA] * 7


def _make_sc1():
    mesh = plsc.VectorSubcoreMesh(core_axis_name="c", subcore_axis_name="s")
    return pl.kernel(
        _sc_edge_pass(True),
        out_type=[
            jax.ShapeDtypeStruct((2, N, HID), jnp.float32),
            jax.ShapeDtypeStruct((2, E, HALF), jnp.float32),
            jax.ShapeDtypeStruct((2, NSUB, 2, HALF), jnp.float32),
        ],
        mesh=mesh,
        scratch_types=(
            _idx_bufs() + _row_bufs(6)
            + [pltpu.VMEM((K, HID), jnp.float32),
               pltpu.VMEM((K, HALF), jnp.float32),
               pltpu.VMEM((2, HALF), jnp.float32),
               pltpu.VMEM_SHARED((N, HID), jnp.float32)]
            + _SEMS
        ),
        compiler_params=pltpu.CompilerParams(use_tc_tiling_on_sc=False),
    )


def _make_sc2():
    mesh = plsc.VectorSubcoreMesh(core_axis_name="c", subcore_axis_name="s")
    return pl.kernel(
        _sc_edge_pass(False),
        out_type=[
            jax.ShapeDtypeStruct((2, N, HID), jnp.float32),
        ],
        mesh=mesh,
        scratch_types=(
            _idx_bufs() + _row_bufs(7)
            + [pltpu.VMEM((K, HID), jnp.float32),
               pltpu.VMEM_SHARED((N, HID), jnp.float32)]
            + _SEMS
        ),
        compiler_params=pltpu.CompilerParams(use_tc_tiling_on_sc=False),
    )


def _tc_tables(h_ref, e_ref, A_ref, B_ref, V_ref, U_ref, C_ref,
               hA_ref, hB_ref, hV_ref, hU_ref, eC_ref):
    h = h_ref[...]
    for w_ref, o_ref in ((A_ref, hA_ref), (B_ref, hB_ref), (V_ref, hV_ref)):
        x = jnp.dot(h, w_ref[...], preferred_element_type=jnp.float32)
        o_ref[0] = x[:, :HALF]
        o_ref[1] = x[:, HALF:]
    hU_ref[...] = jnp.dot(h, U_ref[...], preferred_element_type=jnp.float32)
    ec = jnp.dot(e_ref[...], C_ref[...], preferred_element_type=jnp.float32)
    eC_ref[0] = ec[:, :HALF]
    eC_ref[1] = ec[:, HALF:]


_tables_call = pl.pallas_call(
    _tc_tables,
    out_shape=[
        jax.ShapeDtypeStruct((2, N, HALF), jnp.float32),
        jax.ShapeDtypeStruct((2, N, HALF), jnp.float32),
        jax.ShapeDtypeStruct((2, N, HALF), jnp.float32),
        jax.ShapeDtypeStruct((N, HID), jnp.float32),
        jax.ShapeDtypeStruct((2, R, HALF), jnp.float32),
    ],
)


def _tc_hupdate(h_ref, hU_ref, acc_ref, h1_ref):
    num = jnp.concatenate([acc_ref[0, :, :HALF], acc_ref[1, :, :HALF]], axis=1)
    den = jnp.concatenate([acc_ref[0, :, HALF:], acc_ref[1, :, HALF:]], axis=1)
    x = hU_ref[...] + num / (den + EPS_AGG)
    mu = jnp.mean(x, axis=0, keepdims=True)
    var = jnp.mean((x - mu) * (x - mu), axis=0, keepdims=True)
    hn = jnp.maximum((x - mu) / jnp.sqrt(var + EPS_BN), 0.0)
    h1_ref[...] = h_ref[...] + hn


_hupdate_call = pl.pallas_call(
    _tc_hupdate,
    out_shape=jax.ShapeDtypeStruct((N, HID), jnp.float32),
)

BM = 512


def _tc_edge_mm(mu_ref, inv_ref, C_ref, eh_ref, z_ref):
    lo = eh_ref[0]
    hi = eh_ref[1]
    t_lo = jnp.maximum((lo - mu_ref[0:1, :HALF]) * inv_ref[0:1, :HALF], 0.0)
    t_hi = jnp.maximum((hi - mu_ref[0:1, HALF:]) * inv_ref[0:1, HALF:], 0.0)
    z = (jnp.dot(t_lo, C_ref[:HALF, :], preferred_element_type=jnp.float32)
         + jnp.dot(t_hi, C_ref[HALF:, :], preferred_element_type=jnp.float32))
    z_ref[0] = z[:, :HALF]
    z_ref[1] = z[:, HALF:]


_edge_mm_call = pl.pallas_call(
    _tc_edge_mm,
    grid=(E // BM,),
    in_specs=[
        pl.BlockSpec((1, HID), lambda i: (0, 0)),
        pl.BlockSpec((1, HID), lambda i: (0, 0)),
        pl.BlockSpec((HID, HID), lambda i: (0, 0)),
        pl.BlockSpec((2, BM, HALF), lambda i: (0, i, 0)),
    ],
    out_specs=pl.BlockSpec((2, BM, HALF), lambda i: (0, i, 0)),
    out_shape=jax.ShapeDtypeStruct((2, E, HALF), jnp.float32),
)


def kernel(edge_index, node_id, edge_type, h_emb, e_emb, A, B, C, U, V):
    src = edge_index[0].astype(jnp.int32)
    dst = edge_index[1].astype(jnp.int32)
    et = edge_type.astype(jnp.int32)
    h = h_emb

    sc1 = _make_sc1()
    sc2 = _make_sc2()

    hA0, hB0, hV0, hU0, eC0 = _tables_call(h, e_emb, A[0], B[0], V[0], U[0], C[0])
    acc0, eh0, st0 = sc1(
        src, dst, et,
        hA0.reshape(2 * N, HALF), hB0.reshape(2 * N, HALF),
        hV0.reshape(2 * N, HALF), eC0.reshape(2 * R, HALF))

    ssum = jnp.sum(st0, axis=1)
    mu = jnp.concatenate([ssum[0, 0], ssum[1, 0]]) * (1.0 / E)
    ex2 = jnp.concatenate([ssum[0, 1], ssum[1, 1]]) * (1.0 / E)
    inv = 1.0 / jnp.sqrt(jnp.maximum(ex2 - mu * mu, 0.0) + EPS_BN)
    mu = mu.reshape(1, HID)
    inv = inv.reshape(1, HID)

    h1 = _hupdate_call(h, hU0, acc0)
    hA1, hB1, hV1, hU1, eC1 = _tables_call(h1, e_emb, A[1], B[1], V[1], U[1], C[1])
    z = _edge_mm_call(mu, inv, C[1], eh0)

    (acc1,) = sc2(
        src, dst, et,
        hA1.reshape(2 * N, HALF), hB1.reshape(2 * N, HALF),
        hV1.reshape(2 * N, HALF), eC1.reshape(2 * R, HALF),
        z)
    return _hupdate_call(h1, hU1, acc1)


# speedup vs baseline: 1.7123x; 1.7011x over previous
import jax
import jax.numpy as jnp
from jax import lax
from jax.experimental import pallas as pl
from jax.experimental.pallas import tpu as pltpu
from jax.experimental.pallas import tpu_sc as plsc

N = 10000
E = 320000
R = 16
HID = 128
HALF = 64
NSUB = 16
EPW = E // NSUB
K = 80
NCH = EPW // K
EPS_BN = 1e-5
EPS_AGG = 1e-6
G = HALF // 16


def _sigmoid16(x):
    return 1.0 / (1.0 + jnp.exp(-x))


def _sc_edge_pass(pass1):
    def body(*refs):
        if pass1:
            (src_h, dst_h, et_h, hA, hB, hV, eC,
             num_o, den_o, eh_o, st_o,
             is0, id0, ie0, is1, id1, ie1,
             as0, ad0, ae0, as1, ad1, ae1,
             av0, av1, bv0, bv1, ev, vv,
             sig_v, msg_v, eh_v, st_v,
             num_sh, den_sh, sg0, sg1, se, si0, si1, sh) = refs
            z_h = None
            zv = None
        else:
            (src_h, dst_h, et_h, hA, hB, hV, eC, z_h,
             num_o, den_o,
             is0, id0, ie0, is1, id1, ie1,
             as0, ad0, ae0, as1, ad1, ae1,
             av0, av1, bv0, bv1, ev, vv, zv,
             sig_v, msg_v,
             num_sh, den_sh, sg0, sg1, se, si0, si1, sh) = refs

        isrc, idst, iet = (is0, is1), (id0, id1), (ie0, ie1)
        asrc, adst, aet = (as0, as1), (ad0, ad1), (ae0, ae1)
        av, bv = (av0, av1), (bv0, bv1)
        semg, semi = (sg0, sg1), (si0, si1)

        c = lax.axis_index("c")
        s = lax.axis_index("s")
        base = s * EPW
        coff = c * N
        eoff = c * R

        zero = jnp.zeros((16,), jnp.float32)

        def zb_body(i, _):
            for g in range(G):
                sig_v[i, pl.ds(g * 16, 16)] = zero
            return 0
        lax.fori_loop(0, 40, zb_body, 0)

        @pl.when(s < 10)
        def _zero_shared():
            zrows = sig_v.at[pl.ds(0, 40)]
            for r5 in range(25):
                rows = pl.ds(s * 1000 + r5 * 40, 40)
                pltpu.sync_copy(zrows, num_sh.at[rows])
                pltpu.sync_copy(zrows, den_sh.at[rows])
        plsc.subcore_barrier()

        def fire_idx(j, p):
            off = base + j * K
            pltpu.async_copy(src_h.at[pl.ds(off, K)], isrc[p], semi[p])
            pltpu.async_copy(dst_h.at[pl.ds(off, K)], idst[p], semi[p])
            pltpu.async_copy(et_h.at[pl.ds(off, K)], iet[p], semi[p])

        def wait_idx(p):
            pltpu.make_async_copy(src_h.at[pl.ds(0, K)], isrc[p], semi[p]).wait()
            pltpu.make_async_copy(dst_h.at[pl.ds(0, K)], idst[p], semi[p]).wait()
            pltpu.make_async_copy(et_h.at[pl.ds(0, K)], iet[p], semi[p]).wait()

        def adjust(p):
            for q in range(K // 16):
                sl = pl.ds(q * 16, 16)
                asrc[p][sl] = isrc[p][sl] + coff
                adst[p][sl] = idst[p][sl] + coff
                aet[p][sl] = iet[p][sl] + eoff

        def fire_ab(p):
            pltpu.async_copy(hA.at[asrc[p]], av[p], semg[p])
            pltpu.async_copy(hB.at[adst[p]], bv[p], semg[p])

        def wait_ab(p):
            pltpu.make_async_copy(hA.at[asrc[p]], av[p], semg[p]).wait()
            pltpu.make_async_copy(hB.at[adst[p]], bv[p], semg[p]).wait()

        def fire_ev(j, p):
            pltpu.async_copy(eC.at[aet[p]], ev, se)
            pltpu.async_copy(hV.at[asrc[p]], vv, se)
            if not pass1:
                off = base + j * K
                pltpu.async_copy(z_h.at[c, pl.ds(off, K)], zv, se)

        def wait_ev(p):
            pltpu.make_async_copy(eC.at[aet[p]], ev, se).wait()
            pltpu.make_async_copy(hV.at[asrc[p]], vv, se).wait()
            if not pass1:
                pltpu.make_async_copy(z_h.at[c, pl.ds(0, K)], zv, se).wait()

        def wait_eh(j):
            pltpu.make_async_copy(eh_v, eh_o.at[c, pl.ds(0, K)], sh).wait()

        def compute(p, stats):
            a_v, b_v, e_v, v_v, z_v = av[p], bv[p], ev, vv, zv

            def edge2(i2, st):
                out = list(st) if pass1 else st
                for d in range(2):
                    i = i2 * 2 + d
                    for g in range(G):
                        sl = pl.ds(g * 16, 16)
                        eh = a_v[i, sl] + b_v[i, sl] + e_v[i, sl]
                        if not pass1:
                            eh = eh + z_v[i, sl]
                        sg = _sigmoid16(eh)
                        sig_v[i, sl] = sg
                        msg_v[i, sl] = sg * v_v[i, sl]
                        if pass1:
                            eh_v[i, sl] = eh
                            out[g] = out[g] + eh
                            out[G + g] = out[G + g] + eh * eh
                return tuple(out) if pass1 else out
            return lax.fori_loop(0, K // 2, edge2, stats)

        def step(j, p, stats):
            q = 1 - p

            @pl.when(j + 1 < NCH)
            def _prefetch():
                wait_idx(q)
                adjust(q)
                fire_ab(q)

            wait_ab(p)
            wait_ev(p)
            if pass1:
                @pl.when(j > 0)
                def _weh():
                    wait_eh(j - 1)

            stats = compute(p, stats)

            @pl.when(j + 1 < NCH)
            def _fire_ev_next():
                fire_ev(j + 1, q)

            pltpu.sync_copy(msg_v, num_sh.at[idst[p]], add=True)
            pltpu.sync_copy(sig_v, den_sh.at[idst[p]], add=True)
            if pass1:
                pltpu.async_copy(eh_v, eh_o.at[c, pl.ds(base + j * K, K)], sh)

            @pl.when(j + 2 < NCH)
            def _next_idx():
                fire_idx(j + 2, p)
            return stats

        fire_idx(0, 0)
        wait_idx(0)
        adjust(0)
        fire_ab(0)
        fire_ev(0, 0)
        fire_idx(1, 1)

        stats0 = tuple(jnp.zeros((16,), jnp.float32) for _ in range(2 * G)) if pass1 else 0

        def pair(i, stats):
            j = i * 2
            stats = step(j, 0, stats)
            stats = step(j + 1, 1, stats)
            return stats
        stats = lax.fori_loop(0, NCH // 2, pair, stats0)

        if pass1:
            wait_eh(NCH - 1)
            for g in range(G):
                st_v[0, pl.ds(g * 16, 16)] = stats[g]
                st_v[1, pl.ds(g * 16, 16)] = stats[G + g]
            pltpu.sync_copy(st_v, st_o.at[c, s])

        plsc.subcore_barrier()

        @pl.when(s < 10)
        def _copy_out():
            rows = pl.ds(s * 1000, 1000)
            pltpu.sync_copy(num_sh.at[rows], num_o.at[c, rows])
            pltpu.sync_copy(den_sh.at[rows], den_o.at[c, rows])

    return body


def _idx_bufs():
    return [pltpu.VMEM((K,), jnp.int32) for _ in range(12)]


def _row_bufs(n):
    return [pltpu.VMEM((K, HALF), jnp.float32) for _ in range(n)]


_SEMS = [pltpu.SemaphoreType.DMA] * 6


def _make_sc1():
    mesh = plsc.VectorSubcoreMesh(core_axis_name="c", subcore_axis_name="s")
    return pl.kernel(
        _sc_edge_pass(True),
        out_type=[
            jax.ShapeDtypeStruct((2, N, HALF), jnp.float32),
            jax.ShapeDtypeStruct((2, N, HALF), jnp.float32),
            jax.ShapeDtypeStruct((2, E, HALF), jnp.float32),
            jax.ShapeDtypeStruct((2, NSUB, 2, HALF), jnp.float32),
        ],
        mesh=mesh,
        scratch_types=(
            _idx_bufs() + _row_bufs(6)
            + [pltpu.VMEM((K, HALF), jnp.float32),
               pltpu.VMEM((K, HALF), jnp.float32),
               pltpu.VMEM((K, HALF), jnp.float32),
               pltpu.VMEM((2, HALF), jnp.float32),
               pltpu.VMEM_SHARED((N, HALF), jnp.float32),
               pltpu.VMEM_SHARED((N, HALF), jnp.float32)]
            + _SEMS
        ),
        compiler_params=pltpu.CompilerParams(use_tc_tiling_on_sc=False),
    )


def _make_sc2():
    mesh = plsc.VectorSubcoreMesh(core_axis_name="c", subcore_axis_name="s")
    return pl.kernel(
        _sc_edge_pass(False),
        out_type=[
            jax.ShapeDtypeStruct((2, N, HALF), jnp.float32),
            jax.ShapeDtypeStruct((2, N, HALF), jnp.float32),
        ],
        mesh=mesh,
        scratch_types=(
            _idx_bufs() + _row_bufs(7)
            + [pltpu.VMEM((K, HALF), jnp.float32),
               pltpu.VMEM((K, HALF), jnp.float32),
               pltpu.VMEM_SHARED((N, HALF), jnp.float32),
               pltpu.VMEM_SHARED((N, HALF), jnp.float32)]
            + _SEMS
        ),
        compiler_params=pltpu.CompilerParams(use_tc_tiling_on_sc=False),
    )


def _tc_tables(h_ref, e_ref, A_ref, B_ref, V_ref, U_ref, C_ref,
               hA_ref, hB_ref, hV_ref, hU_ref, eC_ref):
    h = h_ref[...]
    for w_ref, o_ref in ((A_ref, hA_ref), (B_ref, hB_ref), (V_ref, hV_ref)):
        x = jnp.dot(h, w_ref[...], preferred_element_type=jnp.float32)
        o_ref[0] = x[:, :HALF]
        o_ref[1] = x[:, HALF:]
    hU_ref[...] = jnp.dot(h, U_ref[...], preferred_element_type=jnp.float32)
    ec = jnp.dot(e_ref[...], C_ref[...], preferred_element_type=jnp.float32)
    eC_ref[0] = ec[:, :HALF]
    eC_ref[1] = ec[:, HALF:]


_tables_call = pl.pallas_call(
    _tc_tables,
    out_shape=[
        jax.ShapeDtypeStruct((2, N, HALF), jnp.float32),
        jax.ShapeDtypeStruct((2, N, HALF), jnp.float32),
        jax.ShapeDtypeStruct((2, N, HALF), jnp.float32),
        jax.ShapeDtypeStruct((N, HID), jnp.float32),
        jax.ShapeDtypeStruct((2, R, HALF), jnp.float32),
    ],
)


def _tc_hupdate(h_ref, hU_ref, num_ref, den_ref, h1_ref):
    num = jnp.concatenate([num_ref[0], num_ref[1]], axis=1)
    den = jnp.concatenate([den_ref[0], den_ref[1]], axis=1)
    x = hU_ref[...] + num / (den + EPS_AGG)
    mu = jnp.mean(x, axis=0, keepdims=True)
    var = jnp.mean((x - mu) * (x - mu), axis=0, keepdims=True)
    hn = jnp.maximum((x - mu) / jnp.sqrt(var + EPS_BN), 0.0)
    h1_ref[...] = h_ref[...] + hn


_hupdate_call = pl.pallas_call(
    _tc_hupdate,
    out_shape=jax.ShapeDtypeStruct((N, HID), jnp.float32),
)

BM = 512


def _tc_edge_mm(mu_ref, inv_ref, C_ref, eh_ref, z_ref):
    lo = eh_ref[0]
    hi = eh_ref[1]
    t_lo = jnp.maximum((lo - mu_ref[0:1, :HALF]) * inv_ref[0:1, :HALF], 0.0)
    t_hi = jnp.maximum((hi - mu_ref[0:1, HALF:]) * inv_ref[0:1, HALF:], 0.0)
    z = (jnp.dot(t_lo, C_ref[:HALF, :], preferred_element_type=jnp.float32)
         + jnp.dot(t_hi, C_ref[HALF:, :], preferred_element_type=jnp.float32))
    z_ref[0] = z[:, :HALF]
    z_ref[1] = z[:, HALF:]


_edge_mm_call = pl.pallas_call(
    _tc_edge_mm,
    grid=(E // BM,),
    in_specs=[
        pl.BlockSpec((1, HID), lambda i: (0, 0)),
        pl.BlockSpec((1, HID), lambda i: (0, 0)),
        pl.BlockSpec((HID, HID), lambda i: (0, 0)),
        pl.BlockSpec((2, BM, HALF), lambda i: (0, i, 0)),
    ],
    out_specs=pl.BlockSpec((2, BM, HALF), lambda i: (0, i, 0)),
    out_shape=jax.ShapeDtypeStruct((2, E, HALF), jnp.float32),
)


def kernel(edge_index, node_id, edge_type, h_emb, e_emb, A, B, C, U, V):
    src = edge_index[0].astype(jnp.int32)
    dst = edge_index[1].astype(jnp.int32)
    et = edge_type.astype(jnp.int32)
    h = h_emb

    sc1 = _make_sc1()
    sc2 = _make_sc2()

    hA0, hB0, hV0, hU0, eC0 = _tables_call(h, e_emb, A[0], B[0], V[0], U[0], C[0])
    num0, den0, eh0, st0 = sc1(
        src, dst, et,
        hA0.reshape(2 * N, HALF), hB0.reshape(2 * N, HALF),
        hV0.reshape(2 * N, HALF), eC0.reshape(2 * R, HALF))

    ssum = jnp.sum(st0, axis=1)
    mu = jnp.concatenate([ssum[0, 0], ssum[1, 0]]) * (1.0 / E)
    ex2 = jnp.concatenate([ssum[0, 1], ssum[1, 1]]) * (1.0 / E)
    inv = 1.0 / jnp.sqrt(jnp.maximum(ex2 - mu * mu, 0.0) + EPS_BN)
    mu = mu.reshape(1, HID)
    inv = inv.reshape(1, HID)

    h1 = _hupdate_call(h, hU0, num0, den0)
    hA1, hB1, hV1, hU1, eC1 = _tables_call(h1, e_emb, A[1], B[1], V[1], U[1], C[1])
    z = _edge_mm_call(mu, inv, C[1], eh0)

    num1, den1 = sc2(
        src, dst, et,
        hA1.reshape(2 * N, HALF), hB1.reshape(2 * N, HALF),
        hV1.reshape(2 * N, HALF), eC1.reshape(2 * R, HALF),
        z)
    return _hupdate_call(h1, hU1, num1, den1)


# speedup vs baseline: 2.2307x; 1.3028x over previous
import jax
import jax.numpy as jnp
from jax import lax
from jax.experimental import pallas as pl
from jax.experimental.pallas import tpu as pltpu
from jax.experimental.pallas import tpu_sc as plsc

N = 10000
E = 320000
R = 16
HID = 128
HALF = 64
NSUB = 16
EPW = E // NSUB
K = 80
NCH = EPW // K
EPS_BN = 1e-5
EPS_AGG = 1e-6
G = HALF // 16


def _sigmoid16(x):
    return 1.0 / (1.0 + jnp.exp(-x))


def _sc_edge_pass(pass1):
    def body(*refs):
        if pass1:
            (src_h, dst_h, et_h, hA, hB, hV, eC,
             num_o, den_o, eh_o, st_o,
             is0, id0, ie0, is1, id1, ie1,
             as0, ad0, ae0, as1, ad1, ae1,
             av0, av1, bv0, bv1, ev, vv,
             sig_v, msg_v, eh_v, st_v,
             num_sh, den_sh, sg0, sg1, se, si0, si1, sh) = refs
            z_h = None
            zv = None
        else:
            (src_h, dst_h, et_h, hA, hB, hV, eC, z_h,
             num_o, den_o,
             is0, id0, ie0, is1, id1, ie1,
             as0, ad0, ae0, as1, ad1, ae1,
             av0, av1, bv0, bv1, ev, vv, zv,
             sig_v, msg_v,
             num_sh, den_sh, sg0, sg1, se, si0, si1, sh) = refs

        isrc, idst, iet = (is0, is1), (id0, id1), (ie0, ie1)
        asrc, adst, aet = (as0, as1), (ad0, ad1), (ae0, ae1)
        av, bv = (av0, av1), (bv0, bv1)
        semg, semi = (sg0, sg1), (si0, si1)

        c = lax.axis_index("c")
        s = lax.axis_index("s")
        base = s * EPW
        coff = c * N
        eoff = c * R

        zero = jnp.zeros((16,), jnp.float32)

        def zb_body(i, _):
            for g in range(G):
                sig_v[i, pl.ds(g * 16, 16)] = zero
            return 0
        lax.fori_loop(0, 40, zb_body, 0)

        @pl.when(s < 10)
        def _zero_shared():
            zrows = sig_v.at[pl.ds(0, 40)]
            for r5 in range(25):
                rows = pl.ds(s * 1000 + r5 * 40, 40)
                pltpu.sync_copy(zrows, num_sh.at[rows])
                pltpu.sync_copy(zrows, den_sh.at[rows])
        plsc.subcore_barrier()

        def fire_idx(j, p):
            off = base + j * K
            pltpu.async_copy(src_h.at[pl.ds(off, K)], isrc[p], semi[p])
            pltpu.async_copy(dst_h.at[pl.ds(off, K)], idst[p], semi[p])
            pltpu.async_copy(et_h.at[pl.ds(off, K)], iet[p], semi[p])

        def wait_idx(p):
            pltpu.make_async_copy(src_h.at[pl.ds(0, K)], isrc[p], semi[p]).wait()
            pltpu.make_async_copy(dst_h.at[pl.ds(0, K)], idst[p], semi[p]).wait()
            pltpu.make_async_copy(et_h.at[pl.ds(0, K)], iet[p], semi[p]).wait()

        def adjust(p):
            for q in range(K // 16):
                sl = pl.ds(q * 16, 16)
                asrc[p][sl] = isrc[p][sl] + coff
                adst[p][sl] = idst[p][sl] + coff
                aet[p][sl] = iet[p][sl] + eoff

        def fire_ab(p):
            pltpu.async_copy(hA.at[asrc[p]], av[p], semg[p])
            pltpu.async_copy(hB.at[adst[p]], bv[p], semg[p])

        def wait_ab(p):
            pltpu.make_async_copy(hA.at[asrc[p]], av[p], semg[p]).wait()
            pltpu.make_async_copy(hB.at[adst[p]], bv[p], semg[p]).wait()

        def fire_ev(j, p):
            pltpu.async_copy(eC.at[aet[p]], ev, se)
            pltpu.async_copy(hV.at[asrc[p]], vv, se)
            if not pass1:
                off = (base + j * K) // 2
                pltpu.async_copy(z_h.at[c, pl.ds(off, K // 2)], zv, se)

        def wait_ev(p):
            pltpu.make_async_copy(eC.at[aet[p]], ev, se).wait()
            pltpu.make_async_copy(hV.at[asrc[p]], vv, se).wait()
            if not pass1:
                pltpu.make_async_copy(z_h.at[c, pl.ds(0, K // 2)], zv, se).wait()

        def wait_eh(j):
            pltpu.make_async_copy(eh_v, eh_o.at[c, pl.ds(0, K // 2)], sh).wait()

        def compute(p, stats):
            a_v, b_v, e_v, v_v, z_v = av[p], bv[p], ev, vv, zv

            def edge2(i2, st):
                out = list(st) if pass1 else st
                for d in range(2):
                    i = i2 * 2 + d
                    for g in range(G):
                        sl = pl.ds(g * 16, 16)
                        slp = pl.ds(d * HALF + g * 16, 16)
                        eh = a_v[i, sl] + b_v[i, sl] + e_v[i, sl]
                        if not pass1:
                            eh = eh + z_v[i2, slp]
                        sg = _sigmoid16(eh)
                        sig_v[i, sl] = sg
                        msg_v[i, sl] = sg * v_v[i, sl]
                        if pass1:
                            eh_v[i2, slp] = eh
                            out[g] = out[g] + eh
                            out[G + g] = out[G + g] + eh * eh
                return tuple(out) if pass1 else out
            return lax.fori_loop(0, K // 2, edge2, stats)

        def step(j, p, stats):
            q = 1 - p

            @pl.when(j + 1 < NCH)
            def _prefetch():
                wait_idx(q)
                adjust(q)
                fire_ab(q)

            wait_ab(p)
            wait_ev(p)
            if pass1:
                @pl.when(j > 0)
                def _weh():
                    wait_eh(j - 1)

            stats = compute(p, stats)

            @pl.when(j + 1 < NCH)
            def _fire_ev_next():
                fire_ev(j + 1, q)

            pltpu.sync_copy(msg_v, num_sh.at[idst[p]], add=True)
            pltpu.sync_copy(sig_v, den_sh.at[idst[p]], add=True)
            if pass1:
                pltpu.async_copy(
                    eh_v, eh_o.at[c, pl.ds((base + j * K) // 2, K // 2)], sh)

            @pl.when(j + 2 < NCH)
            def _next_idx():
                fire_idx(j + 2, p)
            return stats

        fire_idx(0, 0)
        wait_idx(0)
        adjust(0)
        fire_ab(0)
        fire_ev(0, 0)
        fire_idx(1, 1)

        stats0 = tuple(jnp.zeros((16,), jnp.float32) for _ in range(2 * G)) if pass1 else 0

        def pair(i, stats):
            j = i * 2
            stats = step(j, 0, stats)
            stats = step(j + 1, 1, stats)
            return stats
        stats = lax.fori_loop(0, NCH // 2, pair, stats0)

        if pass1:
            wait_eh(NCH - 1)
            for g in range(G):
                st_v[0, pl.ds(g * 16, 16)] = stats[g]
                st_v[1, pl.ds(g * 16, 16)] = stats[G + g]
            pltpu.sync_copy(st_v, st_o.at[c, s])

        plsc.subcore_barrier()

        @pl.when(s < 10)
        def _copy_out():
            rows = pl.ds(s * 1000, 1000)
            pltpu.sync_copy(num_sh.at[rows], num_o.at[c, rows])
            pltpu.sync_copy(den_sh.at[rows], den_o.at[c, rows])

    return body


def _idx_bufs():
    return [pltpu.VMEM((K,), jnp.int32) for _ in range(12)]


def _row_bufs(n):
    return [pltpu.VMEM((K, HALF), jnp.float32) for _ in range(n)]


_SEMS = [pltpu.SemaphoreType.DMA] * 6


def _make_sc1():
    mesh = plsc.VectorSubcoreMesh(core_axis_name="c", subcore_axis_name="s")
    return pl.kernel(
        _sc_edge_pass(True),
        out_type=[
            jax.ShapeDtypeStruct((2, N, HALF), jnp.float32),
            jax.ShapeDtypeStruct((2, N, HALF), jnp.float32),
            jax.ShapeDtypeStruct((2, E // 2, HID), jnp.float32),
            jax.ShapeDtypeStruct((2, NSUB, 2, HALF), jnp.float32),
        ],
        mesh=mesh,
        scratch_types=(
            _idx_bufs() + _row_bufs(6)
            + [pltpu.VMEM((K, HALF), jnp.float32),
               pltpu.VMEM((K, HALF), jnp.float32),
               pltpu.VMEM((K // 2, HID), jnp.float32),
               pltpu.VMEM((2, HALF), jnp.float32),
               pltpu.VMEM_SHARED((N, HALF), jnp.float32),
               pltpu.VMEM_SHARED((N, HALF), jnp.float32)]
            + _SEMS
        ),
        compiler_params=pltpu.CompilerParams(use_tc_tiling_on_sc=False),
    )


def _make_sc2():
    mesh = plsc.VectorSubcoreMesh(core_axis_name="c", subcore_axis_name="s")
    return pl.kernel(
        _sc_edge_pass(False),
        out_type=[
            jax.ShapeDtypeStruct((2, N, HALF), jnp.float32),
            jax.ShapeDtypeStruct((2, N, HALF), jnp.float32),
        ],
        mesh=mesh,
        scratch_types=(
            _idx_bufs() + _row_bufs(6)
            + [pltpu.VMEM((K // 2, HID), jnp.float32),
               pltpu.VMEM((K, HALF), jnp.float32),
               pltpu.VMEM((K, HALF), jnp.float32),
               pltpu.VMEM_SHARED((N, HALF), jnp.float32),
               pltpu.VMEM_SHARED((N, HALF), jnp.float32)]
            + _SEMS
        ),
        compiler_params=pltpu.CompilerParams(use_tc_tiling_on_sc=False),
    )


def _tc_tables(h_ref, e_ref, A_ref, B_ref, V_ref, U_ref, C_ref,
               hA_ref, hB_ref, hV_ref, hU_ref, eC_ref):
    h = h_ref[...]
    for w_ref, o_ref in ((A_ref, hA_ref), (B_ref, hB_ref), (V_ref, hV_ref)):
        x = jnp.dot(h, w_ref[...], preferred_element_type=jnp.float32)
        o_ref[0] = x[:, :HALF]
        o_ref[1] = x[:, HALF:]
    hU_ref[...] = jnp.dot(h, U_ref[...], preferred_element_type=jnp.float32)
    ec = jnp.dot(e_ref[...], C_ref[...], preferred_element_type=jnp.float32)
    eC_ref[0] = ec[:, :HALF]
    eC_ref[1] = ec[:, HALF:]


_tables_call = pl.pallas_call(
    _tc_tables,
    out_shape=[
        jax.ShapeDtypeStruct((2, N, HALF), jnp.float32),
        jax.ShapeDtypeStruct((2, N, HALF), jnp.float32),
        jax.ShapeDtypeStruct((2, N, HALF), jnp.float32),
        jax.ShapeDtypeStruct((N, HID), jnp.float32),
        jax.ShapeDtypeStruct((2, R, HALF), jnp.float32),
    ],
)


def _tc_hupdate(h_ref, hU_ref, num_ref, den_ref, h1_ref):
    num = jnp.concatenate([num_ref[0], num_ref[1]], axis=1)
    den = jnp.concatenate([den_ref[0], den_ref[1]], axis=1)
    x = hU_ref[...] + num / (den + EPS_AGG)
    mu = jnp.mean(x, axis=0, keepdims=True)
    var = jnp.mean((x - mu) * (x - mu), axis=0, keepdims=True)
    hn = jnp.maximum((x - mu) / jnp.sqrt(var + EPS_BN), 0.0)
    h1_ref[...] = h_ref[...] + hn


_hupdate_call = pl.pallas_call(
    _tc_hupdate,
    out_shape=jax.ShapeDtypeStruct((N, HID), jnp.float32),
)

BMH = 1000


def _tc_edge_mm(mup_ref, invp_ref, W_ref, eh_ref, z_ref):
    t0 = jnp.maximum((eh_ref[0] - mup_ref[0:1]) * invp_ref[0:1], 0.0)
    t1 = jnp.maximum((eh_ref[1] - mup_ref[1:2]) * invp_ref[1:2], 0.0)
    z_ref[0] = (jnp.dot(t0, W_ref[0], preferred_element_type=jnp.float32)
                + jnp.dot(t1, W_ref[1], preferred_element_type=jnp.float32))
    z_ref[1] = (jnp.dot(t0, W_ref[2], preferred_element_type=jnp.float32)
                + jnp.dot(t1, W_ref[3], preferred_element_type=jnp.float32))


_edge_mm_call = pl.pallas_call(
    _tc_edge_mm,
    grid=(E // 2 // BMH,),
    in_specs=[
        pl.BlockSpec((2, HID), lambda i: (0, 0)),
        pl.BlockSpec((2, HID), lambda i: (0, 0)),
        pl.BlockSpec((4, HID, HID), lambda i: (0, 0, 0)),
        pl.BlockSpec((2, BMH, HID), lambda i: (0, i, 0)),
    ],
    out_specs=pl.BlockSpec((2, BMH, HID), lambda i: (0, i, 0)),
    out_shape=jax.ShapeDtypeStruct((2, E // 2, HID), jnp.float32),
)


def kernel(edge_index, node_id, edge_type, h_emb, e_emb, A, B, C, U, V):
    src = edge_index[0].astype(jnp.int32)
    dst = edge_index[1].astype(jnp.int32)
    et = edge_type.astype(jnp.int32)
    h = h_emb

    sc1 = _make_sc1()
    sc2 = _make_sc2()

    hA0, hB0, hV0, hU0, eC0 = _tables_call(h, e_emb, A[0], B[0], V[0], U[0], C[0])
    num0, den0, eh0, st0 = sc1(
        src, dst, et,
        hA0.reshape(2 * N, HALF), hB0.reshape(2 * N, HALF),
        hV0.reshape(2 * N, HALF), eC0.reshape(2 * R, HALF))

    ssum = jnp.sum(st0, axis=1)
    mu = ssum[:, 0] * (1.0 / E)
    ex2 = ssum[:, 1] * (1.0 / E)
    inv = 1.0 / jnp.sqrt(jnp.maximum(ex2 - mu * mu, 0.0) + EPS_BN)
    mup = jnp.concatenate([mu, mu], axis=1)
    invp = jnp.concatenate([inv, inv], axis=1)
    zpad = jnp.zeros((HALF, HALF), jnp.float32)
    C1 = C[1]

    def _bd(m):
        return jnp.concatenate(
            [jnp.concatenate([m, zpad], axis=1),
             jnp.concatenate([zpad, m], axis=1)], axis=0)

    W = jnp.stack([_bd(C1[:HALF, :HALF]), _bd(C1[HALF:, :HALF]),
                   _bd(C1[:HALF, HALF:]), _bd(C1[HALF:, HALF:])])

    h1 = _hupdate_call(h, hU0, num0, den0)
    hA1, hB1, hV1, hU1, eC1 = _tables_call(h1, e_emb, A[1], B[1], V[1], U[1], C[1])
    z = _edge_mm_call(mup, invp, W, eh0)

    num1, den1 = sc2(
        src, dst, et,
        hA1.reshape(2 * N, HALF), hB1.reshape(2 * N, HALF),
        hV1.reshape(2 * N, HALF), eC1.reshape(2 * R, HALF),
        z)
    return _hupdate_call(h1, hU1, num1, den1)


# speedup vs baseline: 2.2311x; 1.0002x over previous
import jax
import jax.numpy as jnp
from jax import lax
from jax.experimental import pallas as pl
from jax.experimental.pallas import tpu as pltpu
from jax.experimental.pallas import tpu_sc as plsc

N = 10000
E = 320000
R = 16
HID = 128
HALF = 64
NSUB = 16
EPW = E // NSUB
K = 80
NCH = EPW // K
EPS_BN = 1e-5
EPS_AGG = 1e-6
G = HALF // 16


def _sigmoid16(x):
    return 1.0 / (1.0 + jnp.exp(-x))


def _sc_edge_pass(pass1):
    def body(*refs):
        if pass1:
            (src_h, dst_h, et_h, hA, hB, hV, eC,
             num_o, den_o, eh_o, st_o,
             is0, id0, ie0, is1, id1, ie1,
             as0, ad0, ae0, as1, ad1, ae1,
             av0, av1, bv0, bv1, ev, vv,
             sig_v, msg_v, eh_v, st_v,
             num_sh, den_sh, sg0, sg1, se, si0, si1, sh, ss) = refs
            z_h = None
            zv = None
        else:
            (src_h, dst_h, et_h, hA, hB, hV, eC, z_h,
             num_o, den_o,
             is0, id0, ie0, is1, id1, ie1,
             as0, ad0, ae0, as1, ad1, ae1,
             av0, av1, bv0, bv1, ev, vv, zv,
             sig_v, msg_v,
             num_sh, den_sh, sg0, sg1, se, si0, si1, sh, ss) = refs

        isrc, idst, iet = (is0, is1), (id0, id1), (ie0, ie1)
        asrc, adst, aet = (as0, as1), (ad0, ad1), (ae0, ae1)
        av, bv = (av0, av1), (bv0, bv1)
        semg, semi = (sg0, sg1), (si0, si1)

        c = lax.axis_index("c")
        s = lax.axis_index("s")
        base = s * EPW
        coff = c * N
        eoff = c * R

        zero = jnp.zeros((16,), jnp.float32)

        def zb_body(i, _):
            for g in range(G):
                sig_v[i, pl.ds(g * 16, 16)] = zero
            return 0
        lax.fori_loop(0, 40, zb_body, 0)

        @pl.when(s < 10)
        def _zero_shared():
            zrows = sig_v.at[pl.ds(0, 40)]
            for r5 in range(25):
                rows = pl.ds(s * 1000 + r5 * 40, 40)
                pltpu.sync_copy(zrows, num_sh.at[rows])
                pltpu.sync_copy(zrows, den_sh.at[rows])
        plsc.subcore_barrier()

        def fire_idx(j, p):
            off = base + j * K
            pltpu.async_copy(src_h.at[pl.ds(off, K)], isrc[p], semi[p])
            pltpu.async_copy(dst_h.at[pl.ds(off, K)], idst[p], semi[p])
            pltpu.async_copy(et_h.at[pl.ds(off, K)], iet[p], semi[p])

        def wait_idx(p):
            pltpu.make_async_copy(src_h.at[pl.ds(0, K)], isrc[p], semi[p]).wait()
            pltpu.make_async_copy(dst_h.at[pl.ds(0, K)], idst[p], semi[p]).wait()
            pltpu.make_async_copy(et_h.at[pl.ds(0, K)], iet[p], semi[p]).wait()

        def adjust(p):
            for q in range(K // 16):
                sl = pl.ds(q * 16, 16)
                asrc[p][sl] = isrc[p][sl] + coff
                adst[p][sl] = idst[p][sl] + coff
                aet[p][sl] = iet[p][sl] + eoff

        def fire_ab(p):
            pltpu.async_copy(hA.at[asrc[p]], av[p], semg[p])
            pltpu.async_copy(hB.at[adst[p]], bv[p], semg[p])

        def wait_ab(p):
            pltpu.make_async_copy(hA.at[asrc[p]], av[p], semg[p]).wait()
            pltpu.make_async_copy(hB.at[adst[p]], bv[p], semg[p]).wait()

        def fire_ev(j, p):
            pltpu.async_copy(eC.at[aet[p]], ev, se)
            pltpu.async_copy(hV.at[asrc[p]], vv, se)
            if not pass1:
                off = (base + j * K) // 2
                pltpu.async_copy(z_h.at[c, pl.ds(off, K // 2)], zv, se)

        def wait_ev(p):
            pltpu.make_async_copy(eC.at[aet[p]], ev, se).wait()
            pltpu.make_async_copy(hV.at[asrc[p]], vv, se).wait()
            if not pass1:
                pltpu.make_async_copy(z_h.at[c, pl.ds(0, K // 2)], zv, se).wait()

        def wait_eh(j):
            pltpu.make_async_copy(eh_v, eh_o.at[c, pl.ds(0, K // 2)], sh).wait()

        def compute(p, stats):
            a_v, b_v, e_v, v_v, z_v = av[p], bv[p], ev, vv, zv

            def edge2(i2, st):
                out = list(st) if pass1 else st
                for d in range(2):
                    i = i2 * 2 + d
                    for g in range(G):
                        sl = pl.ds(g * 16, 16)
                        slp = pl.ds(d * HALF + g * 16, 16)
                        eh = a_v[i, sl] + b_v[i, sl] + e_v[i, sl]
                        if not pass1:
                            eh = eh + z_v[i2, slp]
                        sg = _sigmoid16(eh)
                        sig_v[i, sl] = sg
                        msg_v[i, sl] = sg * v_v[i, sl]
                        if pass1:
                            eh_v[i2, slp] = eh
                            out[g] = out[g] + eh
                            out[G + g] = out[G + g] + eh * eh
                return tuple(out) if pass1 else out
            return lax.fori_loop(0, K // 2, edge2, stats)

        def step(j, p, stats):
            q = 1 - p

            @pl.when(j + 1 < NCH)
            def _prefetch():
                wait_idx(q)
                adjust(q)
                fire_ab(q)

            wait_ab(p)
            wait_ev(p)
            if pass1:
                @pl.when(j > 0)
                def _weh():
                    wait_eh(j - 1)

            stats = compute(p, stats)

            @pl.when(j + 1 < NCH)
            def _fire_ev_next():
                fire_ev(j + 1, q)

            dnum = pltpu.async_copy(msg_v, num_sh.at[idst[p]], ss, add=True)
            pltpu.sync_copy(sig_v, den_sh.at[idst[p]], add=True)
            dnum.wait()
            if pass1:
                pltpu.async_copy(
                    eh_v, eh_o.at[c, pl.ds((base + j * K) // 2, K // 2)], sh)

            @pl.when(j + 2 < NCH)
            def _next_idx():
                fire_idx(j + 2, p)
            return stats

        fire_idx(0, 0)
        wait_idx(0)
        adjust(0)
        fire_ab(0)
        fire_ev(0, 0)
        fire_idx(1, 1)

        stats0 = tuple(jnp.zeros((16,), jnp.float32) for _ in range(2 * G)) if pass1 else 0

        def pair(i, stats):
            j = i * 2
            stats = step(j, 0, stats)
            stats = step(j + 1, 1, stats)
            return stats
        stats = lax.fori_loop(0, NCH // 2, pair, stats0)

        if pass1:
            wait_eh(NCH - 1)
            for g in range(G):
                st_v[0, pl.ds(g * 16, 16)] = stats[g]
                st_v[1, pl.ds(g * 16, 16)] = stats[G + g]
            pltpu.sync_copy(st_v, st_o.at[c, s])

        plsc.subcore_barrier()

        @pl.when(s < 10)
        def _copy_out():
            rows = pl.ds(s * 1000, 1000)
            pltpu.sync_copy(num_sh.at[rows], num_o.at[c, rows])
            pltpu.sync_copy(den_sh.at[rows], den_o.at[c, rows])

    return body


def _idx_bufs():
    return [pltpu.VMEM((K,), jnp.int32) for _ in range(12)]


def _row_bufs(n):
    return [pltpu.VMEM((K, HALF), jnp.float32) for _ in range(n)]


_SEMS = [pltpu.SemaphoreType.DMA] * 7


def _make_sc1():
    mesh = plsc.VectorSubcoreMesh(core_axis_name="c", subcore_axis_name="s")
    return pl.kernel(
        _sc_edge_pass(True),
        out_type=[
            jax.ShapeDtypeStruct((2, N, HALF), jnp.float32),
            jax.ShapeDtypeStruct((2, N, HALF), jnp.float32),
            jax.ShapeDtypeStruct((2, E // 2, HID), jnp.float32),
            jax.ShapeDtypeStruct((2, NSUB, 2, HALF), jnp.float32),
        ],
        mesh=mesh,
        scratch_types=(
            _idx_bufs() + _row_bufs(6)
            + [pltpu.VMEM((K, HALF), jnp.float32),
               pltpu.VMEM((K, HALF), jnp.float32),
               pltpu.VMEM((K // 2, HID), jnp.float32),
               pltpu.VMEM((2, HALF), jnp.float32),
               pltpu.VMEM_SHARED((N, HALF), jnp.float32),
               pltpu.VMEM_SHARED((N, HALF), jnp.float32)]
            + _SEMS
        ),
        compiler_params=pltpu.CompilerParams(use_tc_tiling_on_sc=False),
    )


def _make_sc2():
    mesh = plsc.VectorSubcoreMesh(core_axis_name="c", subcore_axis_name="s")
    return pl.kernel(
        _sc_edge_pass(False),
        out_type=[
            jax.ShapeDtypeStruct((2, N, HALF), jnp.float32),
            jax.ShapeDtypeStruct((2, N, HALF), jnp.float32),
        ],
        mesh=mesh,
        scratch_types=(
            _idx_bufs() + _row_bufs(6)
            + [pltpu.VMEM((K // 2, HID), jnp.float32),
               pltpu.VMEM((K, HALF), jnp.float32),
               pltpu.VMEM((K, HALF), jnp.float32),
               pltpu.VMEM_SHARED((N, HALF), jnp.float32),
               pltpu.VMEM_SHARED((N, HALF), jnp.float32)]
            + _SEMS
        ),
        compiler_params=pltpu.CompilerParams(use_tc_tiling_on_sc=False),
    )


def _tc_tables(h_ref, e_ref, A_ref, B_ref, V_ref, U_ref, C_ref,
               hA_ref, hB_ref, hV_ref, hU_ref, eC_ref):
    h = h_ref[...]
    for w_ref, o_ref in ((A_ref, hA_ref), (B_ref, hB_ref), (V_ref, hV_ref)):
        x = jnp.dot(h, w_ref[...], preferred_element_type=jnp.float32)
        o_ref[0] = x[:, :HALF]
        o_ref[1] = x[:, HALF:]
    hU_ref[...] = jnp.dot(h, U_ref[...], preferred_element_type=jnp.float32)
    ec = jnp.dot(e_ref[...], C_ref[...], preferred_element_type=jnp.float32)
    eC_ref[0] = ec[:, :HALF]
    eC_ref[1] = ec[:, HALF:]


_tables_call = pl.pallas_call(
    _tc_tables,
    out_shape=[
        jax.ShapeDtypeStruct((2, N, HALF), jnp.float32),
        jax.ShapeDtypeStruct((2, N, HALF), jnp.float32),
        jax.ShapeDtypeStruct((2, N, HALF), jnp.float32),
        jax.ShapeDtypeStruct((N, HID), jnp.float32),
        jax.ShapeDtypeStruct((2, R, HALF), jnp.float32),
    ],
)


def _tc_hupdate(h_ref, hU_ref, num_ref, den_ref, h1_ref):
    num = jnp.concatenate([num_ref[0], num_ref[1]], axis=1)
    den = jnp.concatenate([den_ref[0], den_ref[1]], axis=1)
    x = hU_ref[...] + num / (den + EPS_AGG)
    mu = jnp.mean(x, axis=0, keepdims=True)
    var = jnp.mean((x - mu) * (x - mu), axis=0, keepdims=True)
    hn = jnp.maximum((x - mu) / jnp.sqrt(var + EPS_BN), 0.0)
    h1_ref[...] = h_ref[...] + hn


_hupdate_call = pl.pallas_call(
    _tc_hupdate,
    out_shape=jax.ShapeDtypeStruct((N, HID), jnp.float32),
)

BMH = 1000


def _tc_edge_mm(mup_ref, invp_ref, W_ref, eh_ref, z_ref):
    t0 = jnp.maximum((eh_ref[0] - mup_ref[0:1]) * invp_ref[0:1], 0.0)
    t1 = jnp.maximum((eh_ref[1] - mup_ref[1:2]) * invp_ref[1:2], 0.0)
    z_ref[0] = (jnp.dot(t0, W_ref[0], preferred_element_type=jnp.float32)
                + jnp.dot(t1, W_ref[1], preferred_element_type=jnp.float32))
    z_ref[1] = (jnp.dot(t0, W_ref[2], preferred_element_type=jnp.float32)
                + jnp.dot(t1, W_ref[3], preferred_element_type=jnp.float32))


_edge_mm_call = pl.pallas_call(
    _tc_edge_mm,
    grid=(E // 2 // BMH,),
    in_specs=[
        pl.BlockSpec((2, HID), lambda i: (0, 0)),
        pl.BlockSpec((2, HID), lambda i: (0, 0)),
        pl.BlockSpec((4, HID, HID), lambda i: (0, 0, 0)),
        pl.BlockSpec((2, BMH, HID), lambda i: (0, i, 0)),
    ],
    out_specs=pl.BlockSpec((2, BMH, HID), lambda i: (0, i, 0)),
    out_shape=jax.ShapeDtypeStruct((2, E // 2, HID), jnp.float32),
)


def kernel(edge_index, node_id, edge_type, h_emb, e_emb, A, B, C, U, V):
    src = edge_index[0].astype(jnp.int32)
    dst = edge_index[1].astype(jnp.int32)
    et = edge_type.astype(jnp.int32)
    h = h_emb

    sc1 = _make_sc1()
    sc2 = _make_sc2()

    hA0, hB0, hV0, hU0, eC0 = _tables_call(h, e_emb, A[0], B[0], V[0], U[0], C[0])
    num0, den0, eh0, st0 = sc1(
        src, dst, et,
        hA0.reshape(2 * N, HALF), hB0.reshape(2 * N, HALF),
        hV0.reshape(2 * N, HALF), eC0.reshape(2 * R, HALF))

    ssum = jnp.sum(st0, axis=1)
    mu = ssum[:, 0] * (1.0 / E)
    ex2 = ssum[:, 1] * (1.0 / E)
    inv = 1.0 / jnp.sqrt(jnp.maximum(ex2 - mu * mu, 0.0) + EPS_BN)
    mup = jnp.concatenate([mu, mu], axis=1)
    invp = jnp.concatenate([inv, inv], axis=1)
    zpad = jnp.zeros((HALF, HALF), jnp.float32)
    C1 = C[1]

    def _bd(m):
        return jnp.concatenate(
            [jnp.concatenate([m, zpad], axis=1),
             jnp.concatenate([zpad, m], axis=1)], axis=0)

    W = jnp.stack([_bd(C1[:HALF, :HALF]), _bd(C1[HALF:, :HALF]),
                   _bd(C1[:HALF, HALF:]), _bd(C1[HALF:, HALF:])])

    h1 = _hupdate_call(h, hU0, num0, den0)
    hA1, hB1, hV1, hU1, eC1 = _tables_call(h1, e_emb, A[1], B[1], V[1], U[1], C[1])
    z = _edge_mm_call(mup, invp, W, eh0)

    num1, den1 = sc2(
        src, dst, et,
        hA1.reshape(2 * N, HALF), hB1.reshape(2 * N, HALF),
        hV1.reshape(2 * N, HALF), eC1.reshape(2 * R, HALF),
        z)
    return _hupdate_call(h1, hU1, num1, den1)
